# baseline (speedup 1.0000x reference)
import functools

import jax
import jax.numpy as jnp
from jax import lax
from jax.experimental import pallas as pl
from jax.experimental.pallas import tpu as pltpu
from jax.experimental.pallas import tpu_sc as plsc

NUM_BINS = 61928
EMBED_DIM = 16
BATCH = 16384
SPAN = 8

_info = plsc.get_sparse_core_info()
NC, NS, L = _info.num_cores, _info.num_subcores, _info.num_lanes
NREP = NC
FEAT_PER_REP = BATCH // NREP
FCHUNK = 1024
NFCHUNK = FEAT_PER_REP // FCHUNK
GROUPS = FCHUNK // L


def _make_kernel():
    mesh = plsc.VectorSubcoreMesh(core_axis_name="c", subcore_axis_name="s")

    @functools.partial(
        pl.kernel,
        mesh=mesh,
        out_type=jax.ShapeDtypeStruct((EMBED_DIM, BATCH), jnp.float32),
        scratch_types=[
            pltpu.VMEM((NUM_BINS,), jnp.float32),
            pltpu.VMEM((2, FCHUNK, SPAN), jnp.int32),
            pltpu.VMEM((FEAT_PER_REP,), jnp.float32),
            pltpu.SemaphoreType.DMA,
            pltpu.SemaphoreType.DMA,
            pltpu.SemaphoreType.DMA,
        ],
        compiler_params=pltpu.CompilerParams(
            use_tc_tiling_on_sc=False, needs_layout_passes=False
        ),
    )
    def k(idx_hbm, tab_hbm, out_hbm, col_v, idx_v, out_v, sem_a, sem_b, sem_c):
        col_id = lax.axis_index("s")
        rep = lax.axis_index("c")
        feat_base = rep * FEAT_PER_REP
        sems = (sem_a, sem_b)

        def idx_copy(chunk, buf):
            return pltpu.make_async_copy(
                idx_hbm.at[pl.ds(feat_base + chunk * FCHUNK, FCHUNK), :],
                idx_v.at[buf],
                sems[buf],
            )

        col_copy = pltpu.make_async_copy(tab_hbm.at[col_id], col_v, sem_c)
        col_copy.start()
        idx_copy(0, 0).start()
        col_copy.wait()

        inv = jnp.float32(1.0 / SPAN)
        iota = lax.iota(jnp.int32, L)
        splats = [jnp.full((L,), s, jnp.int32) for s in range(SPAN)]

        def make_group_body(buf, chunk_of):
            def group_body(g, carry):
                f_vec = iota + g * L
                bins = plsc.load_gather(idx_v.at[buf], [f_vec, splats[0]])
                acc = plsc.load_gather(col_v, [bins])
                for s in range(1, SPAN):
                    bins = plsc.load_gather(idx_v.at[buf], [f_vec, splats[s]])
                    acc = acc + plsc.load_gather(col_v, [bins])
                out_v[pl.ds(chunk_of(g) * FCHUNK + g * L, L)] = acc * inv
                return carry

            return group_body

        def pair_body(c, carry):
            chunk0 = 2 * c
            chunk1 = 2 * c + 1

            idx_copy(chunk1, 1).start()
            idx_copy(chunk0, 0).wait()
            lax.fori_loop(
                0, GROUPS, make_group_body(0, lambda g: chunk0), 0, unroll=False
            )

            @pl.when(c + 1 < NFCHUNK // 2)
            def _():
                idx_copy(chunk1 + 1, 0).start()

            idx_copy(chunk1, 1).wait()
            lax.fori_loop(
                0, GROUPS, make_group_body(1, lambda g: chunk1), 0, unroll=False
            )
            return carry

        lax.fori_loop(0, NFCHUNK // 2, pair_body, 0, unroll=False)

        pltpu.sync_copy(out_v, out_hbm.at[col_id, pl.ds(feat_base, FEAT_PER_REP)])

    return k


_sc_kernel = _make_kernel()


def kernel(bin_idxs, table):
    tab_t = jnp.transpose(table)
    parts = _sc_kernel(bin_idxs.astype(jnp.int32), tab_t)
    return jnp.transpose(parts)


# speedup vs baseline: 1.2046x; 1.2046x over previous
import functools

import jax
import jax.numpy as jnp
from jax import lax
from jax.experimental import pallas as pl
from jax.experimental.pallas import tpu as pltpu
from jax.experimental.pallas import tpu_sc as plsc

NUM_BINS = 61928
EMBED_DIM = 16
BATCH = 16384
SPAN = 8

_info = plsc.get_sparse_core_info()
NC, NS, L = _info.num_cores, _info.num_subcores, _info.num_lanes
NREP = NC
FEAT_PER_REP = BATCH // NREP
FCHUNK = 1024
NFCHUNK = FEAT_PER_REP // FCHUNK
GROUPS = FCHUNK // L


def _make_kernel():
    mesh = plsc.VectorSubcoreMesh(core_axis_name="c", subcore_axis_name="s")

    @functools.partial(
        pl.kernel,
        mesh=mesh,
        out_type=jax.ShapeDtypeStruct((EMBED_DIM, BATCH), jnp.float32),
        scratch_types=[
            pltpu.VMEM((NUM_BINS,), jnp.float32),
            pltpu.VMEM((2, SPAN, FCHUNK), jnp.int32),
            pltpu.VMEM((FEAT_PER_REP,), jnp.float32),
            pltpu.SemaphoreType.DMA,
            pltpu.SemaphoreType.DMA,
            pltpu.SemaphoreType.DMA,
        ],
        compiler_params=pltpu.CompilerParams(
            use_tc_tiling_on_sc=False, needs_layout_passes=False
        ),
    )
    def k(idx_hbm, tab_hbm, out_hbm, col_v, idx_v, out_v, sem_a, sem_b, sem_c):
        col_id = lax.axis_index("s")
        rep = lax.axis_index("c")
        feat_base = rep * FEAT_PER_REP
        sems = (sem_a, sem_b)

        def idx_copy(chunk, buf):
            return pltpu.make_async_copy(
                idx_hbm.at[:, pl.ds(feat_base + chunk * FCHUNK, FCHUNK)],
                idx_v.at[buf],
                sems[buf],
            )

        col_copy = pltpu.make_async_copy(tab_hbm.at[col_id], col_v, sem_c)
        col_copy.start()
        idx_copy(0, 0).start()
        col_copy.wait()

        inv = jnp.float32(1.0 / SPAN)

        def make_group_body(buf, chunk_of):
            def group_body(g, carry):
                f0 = g * L
                bins = idx_v[buf, 0, pl.ds(f0, L)]
                acc = plsc.load_gather(col_v, [bins])
                for s in range(1, SPAN):
                    bins = idx_v[buf, s, pl.ds(f0, L)]
                    acc = acc + plsc.load_gather(col_v, [bins])
                out_v[pl.ds(chunk_of(g) * FCHUNK + f0, L)] = acc * inv
                return carry

            return group_body

        def pair_body(c, carry):
            chunk0 = 2 * c
            chunk1 = 2 * c + 1

            idx_copy(chunk1, 1).start()
            idx_copy(chunk0, 0).wait()
            lax.fori_loop(
                0, GROUPS, make_group_body(0, lambda g: chunk0), 0, unroll=False
            )

            @pl.when(c + 1 < NFCHUNK // 2)
            def _():
                idx_copy(chunk1 + 1, 0).start()

            idx_copy(chunk1, 1).wait()
            lax.fori_loop(
                0, GROUPS, make_group_body(1, lambda g: chunk1), 0, unroll=False
            )
            return carry

        lax.fori_loop(0, NFCHUNK // 2, pair_body, 0, unroll=False)

        pltpu.sync_copy(out_v, out_hbm.at[col_id, pl.ds(feat_base, FEAT_PER_REP)])

    return k


_sc_kernel = _make_kernel()


def kernel(bin_idxs, table):
    idx_t = jnp.transpose(bin_idxs.astype(jnp.int32))
    tab_t = jnp.transpose(table)
    parts = _sc_kernel(idx_t, tab_t)
    return jnp.transpose(parts)


# speedup vs baseline: 1.2688x; 1.0533x over previous
import functools

import jax
import jax.numpy as jnp
from jax import lax
from jax.experimental import pallas as pl
from jax.experimental.pallas import tpu as pltpu
from jax.experimental.pallas import tpu_sc as plsc

NUM_BINS = 61928
EMBED_DIM = 16
BATCH = 16384
SPAN = 8

_info = plsc.get_sparse_core_info()
NC, NS, L = _info.num_cores, _info.num_subcores, _info.num_lanes
NREP = NC
FEAT_PER_REP = BATCH // NREP
FCHUNK = 1024
NFCHUNK = FEAT_PER_REP // FCHUNK
GROUPS = FCHUNK // L


def _make_kernel():
    mesh = plsc.VectorSubcoreMesh(core_axis_name="c", subcore_axis_name="s")

    @functools.partial(
        pl.kernel,
        mesh=mesh,
        out_type=jax.ShapeDtypeStruct((EMBED_DIM, BATCH), jnp.float32),
        scratch_types=[
            pltpu.VMEM((NUM_BINS,), jnp.float32),
            pltpu.VMEM((2, SPAN, FCHUNK), jnp.int32),
            pltpu.VMEM((FEAT_PER_REP,), jnp.float32),
            pltpu.SemaphoreType.DMA,
            pltpu.SemaphoreType.DMA,
            pltpu.SemaphoreType.DMA,
        ],
        compiler_params=pltpu.CompilerParams(
            use_tc_tiling_on_sc=False, needs_layout_passes=False
        ),
    )
    def k(idx_hbm, tab_hbm, out_hbm, col_v, idx_v, out_v, sem_a, sem_b, sem_c):
        col_id = lax.axis_index("s")
        rep = lax.axis_index("c")
        feat_base = rep * FEAT_PER_REP
        sems = (sem_a, sem_b)

        def idx_copy(chunk, buf):
            return pltpu.make_async_copy(
                idx_hbm.at[:, pl.ds(feat_base + chunk * FCHUNK, FCHUNK)],
                idx_v.at[buf],
                sems[buf],
            )

        col_copy = pltpu.make_async_copy(tab_hbm.at[col_id], col_v, sem_c)
        col_copy.start()
        idx_copy(0, 0).start()
        col_copy.wait()

        inv = jnp.float32(1.0 / SPAN)

        def run_groups(buf, chunk):
            out_base = chunk * FCHUNK

            @plsc.parallel_loop(0, GROUPS, unroll=2)
            def _groups(g):
                f0 = g * L
                bins = idx_v[buf, 0, pl.ds(f0, L)]
                acc = plsc.load_gather(col_v, [bins])
                for s in range(1, SPAN):
                    bins = idx_v[buf, s, pl.ds(f0, L)]
                    acc = acc + plsc.load_gather(col_v, [bins])
                out_v[pl.ds(out_base + f0, L)] = acc * inv

        def pair_body(c, carry):
            chunk0 = 2 * c
            chunk1 = 2 * c + 1

            idx_copy(chunk1, 1).start()
            idx_copy(chunk0, 0).wait()
            run_groups(0, chunk0)

            @pl.when(c + 1 < NFCHUNK // 2)
            def _():
                idx_copy(chunk1 + 1, 0).start()

            idx_copy(chunk1, 1).wait()
            run_groups(1, chunk1)
            return carry

        lax.fori_loop(0, NFCHUNK // 2, pair_body, 0, unroll=False)

        pltpu.sync_copy(out_v, out_hbm.at[col_id, pl.ds(feat_base, FEAT_PER_REP)])

    return k


_sc_kernel = _make_kernel()


def kernel(bin_idxs, table):
    idx_t = jnp.transpose(bin_idxs.astype(jnp.int32))
    tab_t = jnp.transpose(table)
    parts = _sc_kernel(idx_t, tab_t)
    return jnp.transpose(parts)


# speedup vs baseline: 1.2706x; 1.0014x over previous
import functools

import jax
import jax.numpy as jnp
from jax import lax
from jax.experimental import pallas as pl
from jax.experimental.pallas import tpu as pltpu
from jax.experimental.pallas import tpu_sc as plsc

NUM_BINS = 61928
EMBED_DIM = 16
BATCH = 16384
SPAN = 8

_info = plsc.get_sparse_core_info()
NC, NS, L = _info.num_cores, _info.num_subcores, _info.num_lanes
NREP = NC
FEAT_PER_REP = BATCH // NREP
FCHUNK = 1024
NFCHUNK = FEAT_PER_REP // FCHUNK
GROUPS = FCHUNK // L


def _make_kernel():
    mesh = plsc.VectorSubcoreMesh(core_axis_name="c", subcore_axis_name="s")

    @functools.partial(
        pl.kernel,
        mesh=mesh,
        out_type=jax.ShapeDtypeStruct((EMBED_DIM, BATCH), jnp.float32),
        scratch_types=[
            pltpu.VMEM((NUM_BINS,), jnp.float32),
            pltpu.VMEM((2, SPAN, FCHUNK), jnp.int32),
            pltpu.VMEM((FEAT_PER_REP,), jnp.float32),
            pltpu.SemaphoreType.DMA,
            pltpu.SemaphoreType.DMA,
            pltpu.SemaphoreType.DMA,
        ],
        compiler_params=pltpu.CompilerParams(
            use_tc_tiling_on_sc=False, needs_layout_passes=False
        ),
    )
    def k(idx_hbm, tab_hbm, out_hbm, col_v, idx_v, out_v, sem_a, sem_b, sem_c):
        col_id = lax.axis_index("s")
        rep = lax.axis_index("c")
        feat_base = rep * FEAT_PER_REP
        sems = (sem_a, sem_b)

        def idx_copy(chunk, buf):
            return pltpu.make_async_copy(
                idx_hbm.at[:, pl.ds(feat_base + chunk * FCHUNK, FCHUNK)],
                idx_v.at[buf],
                sems[buf],
            )

        col_copy = pltpu.make_async_copy(tab_hbm.at[col_id], col_v, sem_c)
        col_copy.start()
        idx_copy(0, 0).start()
        col_copy.wait()

        inv = jnp.float32(1.0 / SPAN)

        def run_groups(buf, chunk):
            out_base = chunk * FCHUNK

            @plsc.parallel_loop(0, GROUPS, unroll=4)
            def _groups(g):
                f0 = g * L
                bins = idx_v[buf, 0, pl.ds(f0, L)]
                acc = plsc.load_gather(col_v, [bins])
                for s in range(1, SPAN):
                    bins = idx_v[buf, s, pl.ds(f0, L)]
                    acc = acc + plsc.load_gather(col_v, [bins])
                out_v[pl.ds(out_base + f0, L)] = acc * inv

        def pair_body(c, carry):
            chunk0 = 2 * c
            chunk1 = 2 * c + 1

            idx_copy(chunk1, 1).start()
            idx_copy(chunk0, 0).wait()
            run_groups(0, chunk0)

            @pl.when(c + 1 < NFCHUNK // 2)
            def _():
                idx_copy(chunk1 + 1, 0).start()

            idx_copy(chunk1, 1).wait()
            run_groups(1, chunk1)
            return carry

        lax.fori_loop(0, NFCHUNK // 2, pair_body, 0, unroll=False)

        pltpu.sync_copy(out_v, out_hbm.at[col_id, pl.ds(feat_base, FEAT_PER_REP)])

    return k


_sc_kernel = _make_kernel()


def kernel(bin_idxs, table):
    idx_t = jnp.transpose(bin_idxs.astype(jnp.int32))
    tab_t = jnp.transpose(table)
    parts = _sc_kernel(idx_t, tab_t)
    return jnp.transpose(parts)
